# Initial kernel scaffold; baseline (speedup 1.0000x reference)
import jax
import jax.numpy as jnp
from jax.experimental import pallas as pl


def kernel(xw, spatial_ei, spatial_ea, in_w, in_b, q_w, q_b, k_w, k_b, v_w, v_b, film_w, film_b, out_w, out_b, ln_g, ln_b, head_w, head_b):
    raise NotImplementedError("write your pallas kernel here")


# speedup vs baseline: 4.9693x; 4.9693x over previous
import functools

import jax
import jax.numpy as jnp
import numpy as np
from jax.experimental import pallas as pl
from jax.experimental.pallas import tpu as pltpu

ST_HIDDEN = 128
ST_HEADS = 4
ST_LAYERS = 2
RES_SCALE = 0.5
HIDDEN = 128
DK = HIDDEN // ST_HEADS
FIN = 128
E_DIM = 16
INV_SQRT_DK = 1.0 / np.sqrt(DK)


def _mm_bias_body(x_ref, w_ref, b_ref, o_ref):
    o_ref[...] = (
        jnp.dot(x_ref[...], w_ref[...], preferred_element_type=jnp.float32)
        + b_ref[...]
    )


def _mm_bias(x, w, b, block_rows):
    rows = x.shape[0]
    assert rows % block_rows == 0
    grid = rows // block_rows
    return pl.pallas_call(
        _mm_bias_body,
        grid=(grid,),
        in_specs=[
            pl.BlockSpec((block_rows, x.shape[1]), lambda i: (i, 0)),
            pl.BlockSpec((w.shape[0], w.shape[1]), lambda i: (0, 0)),
            pl.BlockSpec((1, w.shape[1]), lambda i: (0, 0)),
        ],
        out_specs=pl.BlockSpec((block_rows, w.shape[1]), lambda i: (i, 0)),
        out_shape=jax.ShapeDtypeStruct((rows, w.shape[1]), jnp.float32),
    )(x, w, b.reshape(1, -1))


def _qkv_body(x_ref, qw_ref, kw_ref, vw_ref, qb_ref, kb_ref, vb_ref,
              q_ref, k_ref, v_ref):
    x = x_ref[...]
    q_ref[...] = jnp.dot(x, qw_ref[...], preferred_element_type=jnp.float32) + qb_ref[...]
    k_ref[...] = jnp.dot(x, kw_ref[...], preferred_element_type=jnp.float32) + kb_ref[...]
    v_ref[...] = jnp.dot(x, vw_ref[...], preferred_element_type=jnp.float32) + vb_ref[...]


def _qkv(x, qw, qb, kw, kb, vw, vb, block_rows):
    rows = x.shape[0]
    grid = rows // block_rows
    wspec = pl.BlockSpec((HIDDEN, HIDDEN), lambda i: (0, 0))
    bspec = pl.BlockSpec((1, HIDDEN), lambda i: (0, 0))
    rspec = pl.BlockSpec((block_rows, HIDDEN), lambda i: (i, 0))
    return pl.pallas_call(
        _qkv_body,
        grid=(grid,),
        in_specs=[rspec, wspec, wspec, wspec, bspec, bspec, bspec],
        out_specs=[rspec, rspec, rspec],
        out_shape=[jax.ShapeDtypeStruct((rows, HIDDEN), jnp.float32)] * 3,
    )(x, qw, kw, vw, qb.reshape(1, -1), kb.reshape(1, -1), vb.reshape(1, -1))


def _outln_body(agg_ref, x_ref, ow_ref, ob_ref, g_ref, b_ref, o_ref):
    y = jnp.dot(agg_ref[...], ow_ref[...], preferred_element_type=jnp.float32) + ob_ref[...]
    h = x_ref[...] + y
    mu = jnp.mean(h, axis=-1, keepdims=True)
    var = jnp.mean((h - mu) ** 2, axis=-1, keepdims=True)
    blk = (h - mu) * jax.lax.rsqrt(var + 1e-5) * g_ref[...] + b_ref[...]
    o_ref[...] = (1.0 - RES_SCALE) * x_ref[...] + RES_SCALE * blk


def _outln(agg, x, ow, ob, g, b, block_rows):
    rows = x.shape[0]
    grid = rows // block_rows
    rspec = pl.BlockSpec((block_rows, ST_HIDDEN), lambda i: (i, 0))
    return pl.pallas_call(
        _outln_body,
        grid=(grid,),
        in_specs=[
            pl.BlockSpec((block_rows, HIDDEN), lambda i: (i, 0)),
            rspec,
            pl.BlockSpec((HIDDEN, ST_HIDDEN), lambda i: (0, 0)),
            pl.BlockSpec((1, ST_HIDDEN), lambda i: (0, 0)),
            pl.BlockSpec((1, ST_HIDDEN), lambda i: (0, 0)),
            pl.BlockSpec((1, ST_HIDDEN), lambda i: (0, 0)),
        ],
        out_specs=rspec,
        out_shape=jax.ShapeDtypeStruct((rows, ST_HIDDEN), jnp.float32),
    )(agg, x, ow, ob.reshape(1, -1), g.reshape(1, -1), b.reshape(1, -1))


def _head_body(x_ref, w_ref, b_ref, o_ref):
    y = jnp.dot(x_ref[...], w_ref[...], preferred_element_type=jnp.float32) + b_ref[0, 0]
    o_ref[...] = jnp.log1p(jnp.exp(-jnp.abs(y))) + jnp.maximum(y, 0.0)


def _head(x, w, b, block_rows):
    rows = x.shape[0]
    grid = rows // block_rows
    return pl.pallas_call(
        _head_body,
        grid=(grid,),
        in_specs=[
            pl.BlockSpec((block_rows, ST_HIDDEN), lambda i: (i, 0)),
            pl.BlockSpec((ST_HIDDEN, 1), lambda i: (0, 0)),
            pl.BlockSpec((1, 1), lambda i: (0, 0), memory_space=pltpu.SMEM),
        ],
        out_specs=pl.BlockSpec((block_rows, 1), lambda i: (i, 0)),
        out_shape=jax.ShapeDtypeStruct((rows, 1), jnp.float32),
    )(x, w, b.reshape(1, 1))


def _edge_phase(Q, K, V, G, B, src, dst, aidx, NT):
    q_e = Q[dst]
    k_e = K[src]
    v_e = V[src]
    gamma = G[aidx]
    beta = B[aidx]
    k_e = k_e * (1.0 + gamma) + beta
    logits = (
        (q_e * k_e).reshape(-1, ST_HEADS, DK).sum(-1) * INV_SQRT_DK
    )
    m = jax.ops.segment_max(logits, dst, num_segments=NT + 8)
    ex = jnp.exp(logits - m[dst])
    den = jax.ops.segment_sum(ex, dst, num_segments=NT + 8)
    att = ex / (den[dst] + 1e-16)
    msg = att[:, :, None] * v_e.reshape(-1, ST_HEADS, DK)
    agg = jnp.zeros((NT + 8, HIDDEN), jnp.float32).at[dst].add(
        msg.reshape(-1, HIDDEN))
    return agg[:NT]


def kernel(xw, spatial_ei, spatial_ea, in_w, in_b, q_w, q_b, k_w, k_b,
           v_w, v_b, film_w, film_b, out_w, out_b, ln_g, ln_b,
           head_w, head_b):
    B_, W, N, Fd = xw.shape
    NT = W * N
    E = spatial_ei.shape[1]

    offs = (jnp.arange(W, dtype=jnp.int32) * N)
    sp_src = (spatial_ei[0][None, :] + offs[:, None]).reshape(-1)
    sp_dst = (spatial_ei[1][None, :] + offs[:, None]).reshape(-1)
    n_temp = (W - 1) * N
    t_src = jnp.arange(n_temp, dtype=jnp.int32)
    src = jnp.concatenate([sp_src, t_src])
    dst = jnp.concatenate([sp_dst, t_src + N])
    aidx = jnp.concatenate([
        jnp.tile(jnp.arange(E, dtype=jnp.int32), (W,)),
        jnp.full((n_temp,), E, jnp.int32),
    ])
    E_tot = src.shape[0]
    EP = ((E_tot + 4095) // 4096) * 4096
    pad = EP - E_tot
    src = jnp.concatenate([src, jnp.zeros((pad,), jnp.int32)])
    dst = jnp.concatenate([dst, jnp.full((pad,), NT, jnp.int32)])
    aidx = jnp.concatenate([aidx, jnp.full((pad,), E + 1, jnp.int32)])

    mean_row = jnp.mean(spatial_ea, axis=0, keepdims=True)
    EAP = ((E + 2 + 2047) // 2048) * 2048
    ea_ext = jnp.concatenate([
        spatial_ea, mean_row,
        jnp.zeros((EAP - E - 1, E_DIM), jnp.float32),
    ])

    x = _mm_bias(xw.reshape(NT, Fd), in_w, in_b, block_rows=2000)

    for l in range(ST_LAYERS):
        Q, K, V = _qkv(x, q_w[l], q_b[l], k_w[l], k_b[l], v_w[l], v_b[l],
                       block_rows=2000)
        fb = _mm_bias(ea_ext, film_w[l], film_b[l], block_rows=2048)
        G = fb[:, :HIDDEN]
        Bt = fb[:, HIDDEN:]
        agg = _edge_phase(Q, K, V, G, Bt, src, dst, aidx, NT)
        x = _outln(agg, x, out_w[l], out_b[l], ln_g[l], ln_b[l],
                   block_rows=2000)

    out = _head(x[N:], head_w, head_b, block_rows=2000)
    return out.reshape(1, N)


# speedup vs baseline: 20.4774x; 4.1208x over previous
import functools

import jax
import jax.numpy as jnp
import numpy as np
from jax import lax
from jax.experimental import pallas as pl
from jax.experimental.pallas import tpu as pltpu
from jax.experimental.pallas import tpu_sc as plsc

ST_HIDDEN = 128
ST_HEADS = 4
ST_LAYERS = 2
RES_SCALE = 0.5
HIDDEN = 128
DK = HIDDEN // ST_HEADS
FIN = 128
E_DIM = 16
INV_SQRT_DK = 1.0 / np.sqrt(DK)


def _mm_bias_body(x_ref, w_ref, b_ref, o_ref):
    o_ref[...] = (
        jnp.dot(x_ref[...], w_ref[...], preferred_element_type=jnp.float32)
        + b_ref[...]
    )


def _mm_bias(x, w, b, block_rows):
    rows = x.shape[0]
    assert rows % block_rows == 0
    grid = rows // block_rows
    return pl.pallas_call(
        _mm_bias_body,
        grid=(grid,),
        in_specs=[
            pl.BlockSpec((block_rows, x.shape[1]), lambda i: (i, 0)),
            pl.BlockSpec((w.shape[0], w.shape[1]), lambda i: (0, 0)),
            pl.BlockSpec((1, w.shape[1]), lambda i: (0, 0)),
        ],
        out_specs=pl.BlockSpec((block_rows, w.shape[1]), lambda i: (i, 0)),
        out_shape=jax.ShapeDtypeStruct((rows, w.shape[1]), jnp.float32),
    )(x, w, b.reshape(1, -1))


def _qkv_body(x_ref, qw_ref, kw_ref, vw_ref, qb_ref, kb_ref, vb_ref,
              q_ref, k_ref, v_ref):
    x = x_ref[...]
    q_ref[...] = jnp.dot(x, qw_ref[...], preferred_element_type=jnp.float32) + qb_ref[...]
    k_ref[...] = jnp.dot(x, kw_ref[...], preferred_element_type=jnp.float32) + kb_ref[...]
    v_ref[...] = jnp.dot(x, vw_ref[...], preferred_element_type=jnp.float32) + vb_ref[...]


def _qkv(x, qw, qb, kw, kb, vw, vb, block_rows):
    rows = x.shape[0]
    grid = rows // block_rows
    wspec = pl.BlockSpec((HIDDEN, HIDDEN), lambda i: (0, 0))
    bspec = pl.BlockSpec((1, HIDDEN), lambda i: (0, 0))
    rspec = pl.BlockSpec((block_rows, HIDDEN), lambda i: (i, 0))
    return pl.pallas_call(
        _qkv_body,
        grid=(grid,),
        in_specs=[rspec, wspec, wspec, wspec, bspec, bspec, bspec],
        out_specs=[rspec, rspec, rspec],
        out_shape=[jax.ShapeDtypeStruct((rows, HIDDEN), jnp.float32)] * 3,
    )(x, qw, kw, vw, qb.reshape(1, -1), kb.reshape(1, -1), vb.reshape(1, -1))


def _den_reduce_body(d_ref, o_ref):
    o_ref[...] = jnp.sum(d_ref[...], axis=0, keepdims=True)


def _den_reduce(den32, block):
    cols = den32.shape[1]
    grid = cols // block
    return pl.pallas_call(
        _den_reduce_body,
        grid=(grid,),
        in_specs=[pl.BlockSpec((32, block), lambda i: (0, i))],
        out_specs=pl.BlockSpec((1, block), lambda i: (0, i)),
        out_shape=jax.ShapeDtypeStruct((1, cols), jnp.float32),
    )(den32)


def _outln_body(agg_ref, den_ref, x_ref, ow_ref, ob_ref, g_ref, b_ref, o_ref):
    a = agg_ref[...]
    den = den_ref[...]
    msg = jnp.concatenate([
        jnp.where(den[:, h:h + 1] > 0.0,
                  a[:, h * 32:(h + 1) * 32] / (den[:, h:h + 1] + 1e-16),
                  0.0)
        for h in range(ST_HEADS)
    ], axis=1)
    y = jnp.dot(msg, ow_ref[...], preferred_element_type=jnp.float32) + ob_ref[...]
    h = x_ref[...] + y
    mu = jnp.mean(h, axis=-1, keepdims=True)
    var = jnp.mean((h - mu) ** 2, axis=-1, keepdims=True)
    blk = (h - mu) * jax.lax.rsqrt(var + 1e-5) * g_ref[...] + b_ref[...]
    o_ref[...] = (1.0 - RES_SCALE) * x_ref[...] + RES_SCALE * blk


def _outln(agg, den, x, ow, ob, g, b, block_rows):
    rows = x.shape[0]
    grid = rows // block_rows
    rspec = pl.BlockSpec((block_rows, ST_HIDDEN), lambda i: (i, 0))
    return pl.pallas_call(
        _outln_body,
        grid=(grid,),
        in_specs=[
            pl.BlockSpec((block_rows, HIDDEN), lambda i: (i, 0)),
            pl.BlockSpec((block_rows, 4), lambda i: (i, 0)),
            rspec,
            pl.BlockSpec((HIDDEN, ST_HIDDEN), lambda i: (0, 0)),
            pl.BlockSpec((1, ST_HIDDEN), lambda i: (0, 0)),
            pl.BlockSpec((1, ST_HIDDEN), lambda i: (0, 0)),
            pl.BlockSpec((1, ST_HIDDEN), lambda i: (0, 0)),
        ],
        out_specs=rspec,
        out_shape=jax.ShapeDtypeStruct((rows, ST_HIDDEN), jnp.float32),
    )(agg, den, x, ow, ob.reshape(1, -1), g.reshape(1, -1), b.reshape(1, -1))


def _head_body(x_ref, w_ref, b_ref, o_ref):
    y = jnp.dot(x_ref[...], w_ref[...], preferred_element_type=jnp.float32) + b_ref[0, 0]
    o_ref[...] = jnp.log1p(jnp.exp(-jnp.abs(y))) + jnp.maximum(y, 0.0)


def _head(x, w, b, block_rows, row_offset):
    rows = x.shape[0] - row_offset
    grid = rows // block_rows
    off = row_offset // block_rows
    return pl.pallas_call(
        _head_body,
        grid=(grid,),
        in_specs=[
            pl.BlockSpec((block_rows, ST_HIDDEN), lambda i: (i + off, 0)),
            pl.BlockSpec((ST_HIDDEN, 1), lambda i: (0, 0)),
            pl.BlockSpec((1, 1), lambda i: (0, 0), memory_space=pltpu.SMEM),
        ],
        out_specs=pl.BlockSpec((block_rows, 1), lambda i: (i, 0)),
        out_shape=jax.ShapeDtypeStruct((rows, 1), jnp.float32),
    )(x, w, b.reshape(1, 1))


_C = 128
_NW = 32
_NTP = 20480
_DEN = 86016


def _sc_mesh():
    return plsc.VectorSubcoreMesh(core_axis_name="c", subcore_axis_name="s",
                                  num_cores=2, num_subcores=16)


def _edge_logits(Q, K, GB, dstg, srcg, aidx, EP):
    ept = EP // _NW
    nch = ept // _C

    @functools.partial(
        pl.kernel,
        out_type=[jax.ShapeDtypeStruct((EP * 16,), jnp.float32),
                  jax.ShapeDtypeStruct((_NW, 16), jnp.float32)],
        mesh=_sc_mesh(),
        compiler_params=pltpu.CompilerParams(needs_layout_passes=False),
        scratch_types=[
            pltpu.VMEM((_C,), jnp.int32),
            pltpu.VMEM((_C,), jnp.int32),
            pltpu.VMEM((_C,), jnp.int32),
            pltpu.VMEM((_C, 128), jnp.float32),
            pltpu.VMEM((_C, 128), jnp.float32),
            pltpu.VMEM((_C, 256), jnp.float32),
            pltpu.VMEM((_C * 16,), jnp.float32),
            pltpu.VMEM((16,), jnp.float32),
            pltpu.SemaphoreType.DMA,
        ],
    )
    def pa(q_hbm, k_hbm, gb_hbm, dst_hbm, src_hbm, aidx_hbm,
           log_hbm, max_hbm,
           dst_v, src_v, aidx_v, q_v, k_v, gb_v, log_v, max_v, sem):
        wid = lax.axis_index("c") * 16 + lax.axis_index("s")
        base = wid * ept
        iota = lax.iota(jnp.int32, 16)
        neg = jnp.full((16,), -1e30, jnp.float32)

        def chunk_body(i, runmax):
            eb = base + i * _C
            pltpu.sync_copy(dst_hbm.at[pl.ds(eb, _C)], dst_v)
            pltpu.sync_copy(src_hbm.at[pl.ds(eb, _C)], src_v)
            pltpu.sync_copy(aidx_hbm.at[pl.ds(eb, _C)], aidx_v)
            cp1 = pltpu.async_copy(q_hbm.at[dst_v], q_v, sem)
            cp2 = pltpu.async_copy(k_hbm.at[src_v], k_v, sem)
            cp3 = pltpu.async_copy(gb_hbm.at[aidx_v], gb_v, sem)
            cp1.wait()
            cp2.wait()
            cp3.wait()

            def edge_body(e, rm):
                t = []
                for j in range(8):
                    sl = pl.ds(j * 16, 16)
                    qj = q_v[e, sl]
                    kj = k_v[e, sl]
                    gj = gb_v[e, sl]
                    bj = gb_v[e, pl.ds(128 + j * 16, 16)]
                    t.append(qj * (kj + kj * gj + bj))
                vec = neg
                for h in range(ST_HEADS):
                    s = t[2 * h] + t[2 * h + 1]
                    lh = jnp.sum(s) * INV_SQRT_DK
                    vec = jnp.where(iota == h,
                                    jnp.full((16,), lh, jnp.float32), vec)
                log_v[pl.ds(e * 16, 16)] = vec
                rm = jnp.maximum(rm, vec)
                return rm

            runmax = lax.fori_loop(0, _C, edge_body, runmax)
            pltpu.sync_copy(log_v, log_hbm.at[pl.ds(eb * 16, _C * 16)])
            return runmax

        runmax = lax.fori_loop(0, nch, chunk_body, neg)
        max_v[...] = runmax
        pltpu.sync_copy(max_v, max_hbm.at[wid])

    return pa(Q, K, GB, dstg, srcg, aidx)


def _edge_aggregate(log, maxes, V, srcS, dstS, EP, NT):
    ept = EP // _NW
    nch = ept // _C

    @functools.partial(
        pl.kernel,
        out_type=[jax.ShapeDtypeStruct((_NTP, 128), jnp.float32),
                  jax.ShapeDtypeStruct((_NW, _DEN), jnp.float32),
                  jax.ShapeDtypeStruct((EP * 16,), jnp.float32),
                  jax.ShapeDtypeStruct((_NW, 128), jnp.float32),
                  jax.ShapeDtypeStruct((_NW, 16), jnp.int32)],
        mesh=_sc_mesh(),
        compiler_params=pltpu.CompilerParams(needs_layout_passes=False),
        scratch_types=[
            pltpu.VMEM((_C,), jnp.int32),
            pltpu.VMEM((_C,), jnp.int32),
            pltpu.VMEM((_C, 128), jnp.float32),
            pltpu.VMEM((_C * 16,), jnp.float32),
            pltpu.VMEM((128,), jnp.float32),
            pltpu.VMEM((_NW, 16), jnp.float32),
            pltpu.VMEM((16,), jnp.int32),
            pltpu.VMEM((_DEN,), jnp.float32),
            pltpu.SemaphoreType.DMA,
        ],
    )
    def pb(log_hbm, max_hbm, v_hbm, src_hbm, dst_hbm,
           agg_hbm, den_hbm, exl_hbm, sb_hbm, sbi_hbm,
           src_v, dst_v, v_v, lg_v, stage_v, mx_v, sbi_v, den_v, sem):
        wid = lax.axis_index("c") * 16 + lax.axis_index("s")
        base = wid * ept
        iota = lax.iota(jnp.int32, 16)
        zero16 = jnp.zeros((16,), jnp.float32)

        pltpu.sync_copy(max_hbm, mx_v)
        m = jnp.full((16,), -1e30, jnp.float32)
        for i in range(_NW):
            m = jnp.maximum(m, mx_v[i, pl.ds(0, 16)])
        M = jnp.max(m)

        def zden(r, _):
            den_v[pl.ds(r * 16, 16)] = zero16
            return 0

        lax.fori_loop(0, _DEN // 16, zden, 0)

        def den_chunk(i, _):
            eb = base + i * _C
            pltpu.sync_copy(dst_hbm.at[pl.ds(eb, _C)], dst_v)
            pltpu.sync_copy(log_hbm.at[pl.ds(eb * 16, _C * 16)], lg_v)

            def prep(g, _):
                for r in range(16):
                    s2 = pl.ds((g * 16 + r) * 16, 16)
                    lg_v[s2] = jnp.exp(lg_v[s2] - M)
                return 0

            lax.fori_loop(0, _C // 16, prep, 0)

            def dengrp(g, _):
                dvec = dst_v[pl.ds(g * 16, 16)]
                for r in range(16):
                    row = lg_v[pl.ds((g * 16 + r) * 16, 16)]
                    didx = jnp.full((16,), dvec[r] * 4, jnp.int32) + iota
                    plsc.addupdate_scatter(den_v, [didx], row)
                return 0

            lax.fori_loop(0, _C // 16, dengrp, 0)
            pltpu.sync_copy(lg_v, exl_hbm.at[pl.ds(eb * 16, _C * 16)])
            return 0

        lax.fori_loop(0, nch, den_chunk, 0)
        pltpu.sync_copy(den_v, den_hbm.at[wid])

        pltpu.sync_copy(dst_hbm.at[pl.ds(base, 16)], sbi_v)
        firstd = sbi_v[pl.ds(0, 16)][0]

        @pl.when(wid > 0)
        def _():
            pltpu.sync_copy(dst_hbm.at[pl.ds(base - 16, 16)], sbi_v)

        prev16 = sbi_v[pl.ds(0, 16)]
        prevd = jnp.where(wid > 0, prev16[15], jnp.int32(-1))
        use_sb = firstd == prevd
        sbi_v[pl.ds(0, 16)] = jnp.full(
            (16,), jnp.where(use_sb, firstd, jnp.int32(NT)), jnp.int32)
        pltpu.sync_copy(sbi_v, sbi_hbm.at[wid])

        def flush(cur, acc):
            for j in range(8):
                stage_v[pl.ds(j * 16, 16)] = acc[j]
            to_sb = (cur == firstd) & use_sb

            @pl.when(to_sb)
            def _():
                pltpu.sync_copy(stage_v, sb_hbm.at[wid])

            @pl.when(jnp.logical_not(to_sb))
            def _():
                pltpu.sync_copy(stage_v, agg_hbm.at[cur])

        def chunk_body(i, carry):
            eb = base + i * _C
            pltpu.sync_copy(src_hbm.at[pl.ds(eb, _C)], src_v)
            pltpu.sync_copy(dst_hbm.at[pl.ds(eb, _C)], dst_v)
            cp = pltpu.async_copy(v_hbm.at[src_v], v_v, sem)
            pltpu.sync_copy(exl_hbm.at[pl.ds(eb * 16, _C * 16)], lg_v)
            cp.wait()

            def group_body(g, carry):
                dvec = dst_v[pl.ds(g * 16, 16)]
                for r in range(16):
                    cur, acc = carry[0], list(carry[1:])
                    e = g * 16 + r
                    d = dvec[r]
                    row = lg_v[pl.ds(e * 16, 16)]
                    changed = d != cur

                    @pl.when(changed)
                    def _():
                        flush(cur, acc)

                    msg = []
                    for h in range(ST_HEADS):
                        sh = jnp.full((16,), row[h], jnp.float32)
                        for j in (2 * h, 2 * h + 1):
                            msg.append(v_v[e, pl.ds(j * 16, 16)] * sh)
                    acc = [jnp.where(changed, msg[j], acc[j] + msg[j])
                           for j in range(8)]
                    carry = (d, *acc)
                return carry

            return lax.fori_loop(0, _C // 16, group_body, carry)

        init = (jnp.int32(0) + firstd,) + tuple(zero16 for _ in range(8))
        carry = lax.fori_loop(0, nch, chunk_body, init)
        flush(carry[0], list(carry[1:]))

        @pl.when(jnp.logical_not(use_sb))
        def _():
            for j in range(8):
                stage_v[pl.ds(j * 16, 16)] = zero16
            pltpu.sync_copy(stage_v, sb_hbm.at[wid])

    agg, den32, _exl, sb, sbi = pb(log, maxes, V, srcS, dstS)
    return agg, den32, sb, sbi


def kernel(xw, spatial_ei, spatial_ea, in_w, in_b, q_w, q_b, k_w, k_b,
           v_w, v_b, film_w, film_b, out_w, out_b, ln_g, ln_b,
           head_w, head_b):
    B_, W, N, Fd = xw.shape
    NT = W * N
    E = spatial_ei.shape[1]

    n_temp = (W - 1) * N
    t_src = jnp.arange(n_temp, dtype=jnp.int32)
    offs = jnp.arange(W, dtype=jnp.int32) * N
    sp_src = (spatial_ei[0][None, :] + offs[:, None]).reshape(-1)
    sp_dst = (spatial_ei[1][None, :] + offs[:, None]).reshape(-1)
    src = jnp.concatenate([sp_src, t_src])
    dst = jnp.concatenate([sp_dst, t_src + N])
    aidx = jnp.concatenate([
        jnp.tile(jnp.arange(E, dtype=jnp.int32), (W,)),
        jnp.full((n_temp,), E, jnp.int32),
    ])
    E_tot = src.shape[0]
    EP = ((E_tot + 4095) // 4096) * 4096
    pad = EP - E_tot
    src = jnp.concatenate([src, jnp.zeros((pad,), jnp.int32)])
    dst = jnp.concatenate([dst, jnp.full((pad,), NT, jnp.int32)])
    aidx = jnp.concatenate([aidx, jnp.full((pad,), E + 1, jnp.int32)])
    order = jnp.argsort(dst)
    srcS = src[order]
    dstS = dst[order]
    aidxS = aidx[order]
    dstSg = jnp.minimum(dstS, NT - 1)

    mean_row = jnp.mean(spatial_ea, axis=0, keepdims=True)
    EAP = ((E + 2 + 2047) // 2048) * 2048
    ea_ext = jnp.concatenate([
        spatial_ea, mean_row,
        jnp.zeros((EAP - E - 1, E_DIM), jnp.float32),
    ])

    x = _mm_bias(xw.reshape(NT, Fd), in_w, in_b, block_rows=2000)

    for l in range(ST_LAYERS):
        Q, K, V = _qkv(x, q_w[l], q_b[l], k_w[l], k_b[l], v_w[l], v_b[l],
                       block_rows=2000)
        GB = _mm_bias(ea_ext, film_w[l], film_b[l], block_rows=2048)
        log, maxes = _edge_logits(Q, K, GB, dstSg, srcS, aidxS, EP)
        agg, den32, sb, sbi = _edge_aggregate(log, maxes, V, srcS, dstS,
                                              EP, NT)
        agg = agg.at[sbi[:, 0]].add(sb)
        den = _den_reduce(den32, block=4096).reshape(-1, 4)
        x = _outln(agg, den, x, out_w[l], out_b[l], ln_g[l], ln_b[l],
                   block_rows=2000)

    out = _head(x, head_w, head_b, block_rows=2000, row_offset=N)
    return out.reshape(1, N)


# speedup vs baseline: 21.2132x; 1.0359x over previous
import functools

import jax
import jax.numpy as jnp
import numpy as np
from jax import lax
from jax.experimental import pallas as pl
from jax.experimental.pallas import tpu as pltpu
from jax.experimental.pallas import tpu_sc as plsc

ST_HIDDEN = 128
ST_HEADS = 4
ST_LAYERS = 2
RES_SCALE = 0.5
HIDDEN = 128
DK = HIDDEN // ST_HEADS
FIN = 128
E_DIM = 16
INV_SQRT_DK = 1.0 / np.sqrt(DK)


def _mm_bias_body(x_ref, w_ref, b_ref, o_ref):
    o_ref[...] = (
        jnp.dot(x_ref[...], w_ref[...], preferred_element_type=jnp.float32)
        + b_ref[...]
    )


def _mm_bias(x, w, b, block_rows):
    rows = x.shape[0]
    assert rows % block_rows == 0
    grid = rows // block_rows
    return pl.pallas_call(
        _mm_bias_body,
        grid=(grid,),
        in_specs=[
            pl.BlockSpec((block_rows, x.shape[1]), lambda i: (i, 0)),
            pl.BlockSpec((w.shape[0], w.shape[1]), lambda i: (0, 0)),
            pl.BlockSpec((1, w.shape[1]), lambda i: (0, 0)),
        ],
        out_specs=pl.BlockSpec((block_rows, w.shape[1]), lambda i: (i, 0)),
        out_shape=jax.ShapeDtypeStruct((rows, w.shape[1]), jnp.float32),
    )(x, w, b.reshape(1, -1))


def _qkv_body(x_ref, qw_ref, kw_ref, vw_ref, qb_ref, kb_ref, vb_ref,
              q_ref, k_ref, v_ref):
    x = x_ref[...]
    q_ref[...] = jnp.dot(x, qw_ref[...], preferred_element_type=jnp.float32) + qb_ref[...]
    k_ref[...] = jnp.dot(x, kw_ref[...], preferred_element_type=jnp.float32) + kb_ref[...]
    v_ref[...] = jnp.dot(x, vw_ref[...], preferred_element_type=jnp.float32) + vb_ref[...]


def _qkv(x, qw, qb, kw, kb, vw, vb, block_rows):
    rows = x.shape[0]
    grid = rows // block_rows
    wspec = pl.BlockSpec((HIDDEN, HIDDEN), lambda i: (0, 0))
    bspec = pl.BlockSpec((1, HIDDEN), lambda i: (0, 0))
    rspec = pl.BlockSpec((block_rows, HIDDEN), lambda i: (i, 0))
    return pl.pallas_call(
        _qkv_body,
        grid=(grid,),
        in_specs=[rspec, wspec, wspec, wspec, bspec, bspec, bspec],
        out_specs=[rspec, rspec, rspec],
        out_shape=[jax.ShapeDtypeStruct((rows, HIDDEN), jnp.float32)] * 3,
    )(x, qw, kw, vw, qb.reshape(1, -1), kb.reshape(1, -1), vb.reshape(1, -1))


def _den_reduce_body(d_ref, o_ref):
    o_ref[...] = jnp.sum(d_ref[...], axis=0, keepdims=True)


def _den_reduce(den32, block):
    cols = den32.shape[1]
    grid = cols // block
    return pl.pallas_call(
        _den_reduce_body,
        grid=(grid,),
        in_specs=[pl.BlockSpec((32, block), lambda i: (0, i))],
        out_specs=pl.BlockSpec((1, block), lambda i: (0, i)),
        out_shape=jax.ShapeDtypeStruct((1, cols), jnp.float32),
    )(den32)


def _outln_body(agg_ref, den_ref, x_ref, ow_ref, ob_ref, g_ref, b_ref, o_ref):
    a = agg_ref[...]
    den = den_ref[...]
    msg = jnp.concatenate([
        jnp.where(den[:, h:h + 1] > 0.0,
                  a[:, h * 32:(h + 1) * 32] / (den[:, h:h + 1] + 1e-16),
                  0.0)
        for h in range(ST_HEADS)
    ], axis=1)
    y = jnp.dot(msg, ow_ref[...], preferred_element_type=jnp.float32) + ob_ref[...]
    h = x_ref[...] + y
    mu = jnp.mean(h, axis=-1, keepdims=True)
    var = jnp.mean((h - mu) ** 2, axis=-1, keepdims=True)
    blk = (h - mu) * jax.lax.rsqrt(var + 1e-5) * g_ref[...] + b_ref[...]
    o_ref[...] = (1.0 - RES_SCALE) * x_ref[...] + RES_SCALE * blk


def _outln(agg, den, x, ow, ob, g, b, block_rows):
    rows = x.shape[0]
    grid = rows // block_rows
    rspec = pl.BlockSpec((block_rows, ST_HIDDEN), lambda i: (i, 0))
    return pl.pallas_call(
        _outln_body,
        grid=(grid,),
        in_specs=[
            pl.BlockSpec((block_rows, HIDDEN), lambda i: (i, 0)),
            pl.BlockSpec((block_rows, 4), lambda i: (i, 0)),
            rspec,
            pl.BlockSpec((HIDDEN, ST_HIDDEN), lambda i: (0, 0)),
            pl.BlockSpec((1, ST_HIDDEN), lambda i: (0, 0)),
            pl.BlockSpec((1, ST_HIDDEN), lambda i: (0, 0)),
            pl.BlockSpec((1, ST_HIDDEN), lambda i: (0, 0)),
        ],
        out_specs=rspec,
        out_shape=jax.ShapeDtypeStruct((rows, ST_HIDDEN), jnp.float32),
    )(agg, den, x, ow, ob.reshape(1, -1), g.reshape(1, -1), b.reshape(1, -1))


def _head_body(x_ref, w_ref, b_ref, o_ref):
    y = jnp.dot(x_ref[...], w_ref[...], preferred_element_type=jnp.float32) + b_ref[0, 0]
    o_ref[...] = jnp.log1p(jnp.exp(-jnp.abs(y))) + jnp.maximum(y, 0.0)


def _head(x, w, b, block_rows, row_offset):
    rows = x.shape[0] - row_offset
    grid = rows // block_rows
    off = row_offset // block_rows
    return pl.pallas_call(
        _head_body,
        grid=(grid,),
        in_specs=[
            pl.BlockSpec((block_rows, ST_HIDDEN), lambda i: (i + off, 0)),
            pl.BlockSpec((ST_HIDDEN, 1), lambda i: (0, 0)),
            pl.BlockSpec((1, 1), lambda i: (0, 0), memory_space=pltpu.SMEM),
        ],
        out_specs=pl.BlockSpec((block_rows, 1), lambda i: (i, 0)),
        out_shape=jax.ShapeDtypeStruct((rows, 1), jnp.float32),
    )(x, w, b.reshape(1, 1))


_C = 128
_NW = 32
_NTP = 20480
_DEN = 86016


def _sc_mesh():
    return plsc.VectorSubcoreMesh(core_axis_name="c", subcore_axis_name="s",
                                  num_cores=2, num_subcores=16)


def _edge_logits(Q, K, GB, idx3, EP):
    ept = EP // _NW
    nch = ept // _C

    @functools.partial(
        pl.kernel,
        out_type=[jax.ShapeDtypeStruct((EP * 16,), jnp.float32),
                  jax.ShapeDtypeStruct((_NW, 16), jnp.float32)],
        mesh=_sc_mesh(),
        compiler_params=pltpu.CompilerParams(needs_layout_passes=False),
        scratch_types=[
            pltpu.VMEM((3 * _C,), jnp.int32),
            pltpu.VMEM((_C, 128), jnp.float32),
            pltpu.VMEM((_C, 128), jnp.float32),
            pltpu.VMEM((_C, 256), jnp.float32),
            pltpu.VMEM((_C * 16,), jnp.float32),
            pltpu.VMEM((16,), jnp.float32),
            pltpu.SemaphoreType.DMA,
        ],
    )
    def pa(q_hbm, k_hbm, gb_hbm, idx3_hbm,
           log_hbm, max_hbm,
           idx_v, q_v, k_v, gb_v, log_v, max_v, sem):
        wid = lax.axis_index("c") * 16 + lax.axis_index("s")
        base = wid * ept
        iota = lax.iota(jnp.int32, 16)
        neg = jnp.full((16,), -1e30, jnp.float32)

        def chunk_body(i, runmax):
            eb = base + i * _C
            pltpu.sync_copy(idx3_hbm.at[pl.ds(eb * 3, 3 * _C)], idx_v)
            cp1 = pltpu.async_copy(q_hbm.at[idx_v.at[pl.ds(_C, _C)]],
                                   q_v, sem)
            cp2 = pltpu.async_copy(k_hbm.at[idx_v.at[pl.ds(0, _C)]],
                                   k_v, sem)
            cp3 = pltpu.async_copy(gb_hbm.at[idx_v.at[pl.ds(2 * _C, _C)]],
                                   gb_v, sem)
            cp1.wait()
            cp2.wait()
            cp3.wait()

            def edge_body(e, rm):
                t = []
                for j in range(8):
                    sl = pl.ds(j * 16, 16)
                    qj = q_v[e, sl]
                    kj = k_v[e, sl]
                    gj = gb_v[e, sl]
                    bj = gb_v[e, pl.ds(128 + j * 16, 16)]
                    t.append(qj * (kj + kj * gj + bj))
                vec = neg
                for h in range(ST_HEADS):
                    s = t[2 * h] + t[2 * h + 1]
                    lh = jnp.sum(s) * INV_SQRT_DK
                    vec = jnp.where(iota == h,
                                    jnp.full((16,), lh, jnp.float32), vec)
                log_v[pl.ds(e * 16, 16)] = vec
                rm = jnp.maximum(rm, vec)
                return rm

            runmax = lax.fori_loop(0, _C, edge_body, runmax)
            pltpu.sync_copy(log_v, log_hbm.at[pl.ds(eb * 16, _C * 16)])
            return runmax

        runmax = lax.fori_loop(0, nch, chunk_body, neg)
        max_v[...] = runmax
        pltpu.sync_copy(max_v, max_hbm.at[wid])

    return pa(Q, K, GB, idx3)


def _edge_aggregate(log, maxes, V, idx3, dstS, EP, NT):
    ept = EP // _NW
    nch = ept // _C

    @functools.partial(
        pl.kernel,
        out_type=[jax.ShapeDtypeStruct((_NTP, 128), jnp.float32),
                  jax.ShapeDtypeStruct((_NW, _DEN), jnp.float32),
                  jax.ShapeDtypeStruct((EP * 16,), jnp.float32),
                  jax.ShapeDtypeStruct((_NW, 128), jnp.float32),
                  jax.ShapeDtypeStruct((_NW, 16), jnp.int32)],
        mesh=_sc_mesh(),
        compiler_params=pltpu.CompilerParams(needs_layout_passes=False),
        scratch_types=[
            pltpu.VMEM((3 * _C,), jnp.int32),
            pltpu.VMEM((_C,), jnp.int32),
            pltpu.VMEM((_C, 128), jnp.float32),
            pltpu.VMEM((_C * 16,), jnp.float32),
            pltpu.VMEM((128,), jnp.float32),
            pltpu.VMEM((_NW, 16), jnp.float32),
            pltpu.VMEM((16,), jnp.int32),
            pltpu.VMEM((_DEN,), jnp.float32),
            pltpu.SemaphoreType.DMA,
        ],
    )
    def pb(log_hbm, max_hbm, v_hbm, idx3_hbm, dst_hbm,
           agg_hbm, den_hbm, exl_hbm, sb_hbm, sbi_hbm,
           idx_v, dst_v, v_v, lg_v, stage_v, mx_v, sbi_v, den_v, sem):
        wid = lax.axis_index("c") * 16 + lax.axis_index("s")
        base = wid * ept
        iota = lax.iota(jnp.int32, 16)
        zero16 = jnp.zeros((16,), jnp.float32)

        pltpu.sync_copy(max_hbm, mx_v)
        m = jnp.full((16,), -1e30, jnp.float32)
        for i in range(_NW):
            m = jnp.maximum(m, mx_v[i, pl.ds(0, 16)])
        M = jnp.max(m)

        def zden(r, _):
            den_v[pl.ds(r * 16, 16)] = zero16
            return 0

        lax.fori_loop(0, _DEN // 16, zden, 0)

        def den_chunk(i, _):
            eb = base + i * _C
            pltpu.sync_copy(dst_hbm.at[pl.ds(eb, _C)], dst_v)
            pltpu.sync_copy(log_hbm.at[pl.ds(eb * 16, _C * 16)], lg_v)

            def prep(g, _):
                for r in range(16):
                    s2 = pl.ds((g * 16 + r) * 16, 16)
                    lg_v[s2] = jnp.exp(lg_v[s2] - M)
                return 0

            lax.fori_loop(0, _C // 16, prep, 0)

            def dengrp(g, _):
                dvec = dst_v[pl.ds(g * 16, 16)]
                for r in range(16):
                    row = lg_v[pl.ds((g * 16 + r) * 16, 16)]
                    didx = jnp.full((16,), dvec[r] * 4, jnp.int32) + iota
                    plsc.addupdate_scatter(den_v, [didx], row)
                return 0

            lax.fori_loop(0, _C // 16, dengrp, 0)
            pltpu.sync_copy(lg_v, exl_hbm.at[pl.ds(eb * 16, _C * 16)])
            return 0

        lax.fori_loop(0, nch, den_chunk, 0)
        pltpu.sync_copy(den_v, den_hbm.at[wid])

        pltpu.sync_copy(dst_hbm.at[pl.ds(base, 16)], sbi_v)
        firstd = sbi_v[pl.ds(0, 16)][0]

        @pl.when(wid > 0)
        def _():
            pltpu.sync_copy(dst_hbm.at[pl.ds(base - 16, 16)], sbi_v)

        prev16 = sbi_v[pl.ds(0, 16)]
        prevd = jnp.where(wid > 0, prev16[15], jnp.int32(-1))
        use_sb = firstd == prevd
        sbi_v[pl.ds(0, 16)] = jnp.full(
            (16,), jnp.where(use_sb, firstd, jnp.int32(NT)), jnp.int32)
        pltpu.sync_copy(sbi_v, sbi_hbm.at[wid])

        def flush(cur, acc):
            for j in range(8):
                stage_v[pl.ds(j * 16, 16)] = acc[j]
            to_sb = (cur == firstd) & use_sb

            @pl.when(to_sb)
            def _():
                pltpu.sync_copy(stage_v, sb_hbm.at[wid])

            @pl.when(jnp.logical_not(to_sb))
            def _():
                pltpu.sync_copy(stage_v, agg_hbm.at[cur])

        def chunk_body(i, carry):
            eb = base + i * _C
            pltpu.sync_copy(idx3_hbm.at[pl.ds(eb * 3, 3 * _C)], idx_v)
            pltpu.sync_copy(dst_hbm.at[pl.ds(eb, _C)], dst_v)
            cp = pltpu.async_copy(v_hbm.at[idx_v.at[pl.ds(0, _C)]],
                                  v_v, sem)
            pltpu.sync_copy(exl_hbm.at[pl.ds(eb * 16, _C * 16)], lg_v)
            cp.wait()

            def group_body(g, carry):
                dvec = dst_v[pl.ds(g * 16, 16)]
                for r in range(16):
                    cur, acc = carry[0], list(carry[1:])
                    e = g * 16 + r
                    d = dvec[r]
                    row = lg_v[pl.ds(e * 16, 16)]
                    changed = d != cur

                    @pl.when(changed)
                    def _():
                        flush(cur, acc)

                    msg = []
                    for h in range(ST_HEADS):
                        sh = jnp.full((16,), row[h], jnp.float32)
                        for j in (2 * h, 2 * h + 1):
                            msg.append(v_v[e, pl.ds(j * 16, 16)] * sh)
                    acc = [jnp.where(changed, msg[j], acc[j] + msg[j])
                           for j in range(8)]
                    carry = (d, *acc)
                return carry

            return lax.fori_loop(0, _C // 16, group_body, carry)

        init = (jnp.int32(0) + firstd,) + tuple(zero16 for _ in range(8))
        carry = lax.fori_loop(0, nch, chunk_body, init)
        flush(carry[0], list(carry[1:]))

        @pl.when(jnp.logical_not(use_sb))
        def _():
            for j in range(8):
                stage_v[pl.ds(j * 16, 16)] = zero16
            pltpu.sync_copy(stage_v, sb_hbm.at[wid])

    agg, den32, _exl, sb, sbi = pb(log, maxes, V, idx3, dstS)
    return agg, den32, sb, sbi


def kernel(xw, spatial_ei, spatial_ea, in_w, in_b, q_w, q_b, k_w, k_b,
           v_w, v_b, film_w, film_b, out_w, out_b, ln_g, ln_b,
           head_w, head_b):
    B_, W, N, Fd = xw.shape
    NT = W * N
    E = spatial_ei.shape[1]

    n_temp = (W - 1) * N
    t_src = jnp.arange(n_temp, dtype=jnp.int32)
    offs = jnp.arange(W, dtype=jnp.int32) * N
    sp_src = (spatial_ei[0][None, :] + offs[:, None]).reshape(-1)
    sp_dst = (spatial_ei[1][None, :] + offs[:, None]).reshape(-1)
    src = jnp.concatenate([sp_src, t_src])
    dst = jnp.concatenate([sp_dst, t_src + N])
    aidx = jnp.concatenate([
        jnp.tile(jnp.arange(E, dtype=jnp.int32), (W,)),
        jnp.full((n_temp,), E, jnp.int32),
    ])
    E_tot = src.shape[0]
    EP = ((E_tot + 4095) // 4096) * 4096
    pad = EP - E_tot
    src = jnp.concatenate([src, jnp.zeros((pad,), jnp.int32)])
    dst = jnp.concatenate([dst, jnp.full((pad,), NT, jnp.int32)])
    aidx = jnp.concatenate([aidx, jnp.full((pad,), E + 1, jnp.int32)])
    order = jnp.argsort(dst)
    srcS = src[order]
    dstS = dst[order]
    aidxS = aidx[order]
    dstSg = jnp.minimum(dstS, NT - 1)
    idx3 = jnp.stack([srcS.reshape(-1, 128), dstSg.reshape(-1, 128),
                      aidxS.reshape(-1, 128)], axis=1).reshape(-1)

    mean_row = jnp.mean(spatial_ea, axis=0, keepdims=True)
    EAP = ((E + 2 + 2047) // 2048) * 2048
    ea_ext = jnp.concatenate([
        spatial_ea, mean_row,
        jnp.zeros((EAP - E - 1, E_DIM), jnp.float32),
    ])

    x = _mm_bias(xw.reshape(NT, Fd), in_w, in_b, block_rows=2000)

    for l in range(ST_LAYERS):
        Q, K, V = _qkv(x, q_w[l], q_b[l], k_w[l], k_b[l], v_w[l], v_b[l],
                       block_rows=2000)
        GB = _mm_bias(ea_ext, film_w[l], film_b[l], block_rows=2048)
        log, maxes = _edge_logits(Q, K, GB, idx3, EP)
        agg, den32, sb, sbi = _edge_aggregate(log, maxes, V, idx3, dstS,
                                              EP, NT)
        agg = agg.at[sbi[:, 0]].add(sb)
        den = _den_reduce(den32, block=4096).reshape(-1, 4)
        x = _outln(agg, den, x, out_w[l], out_b[l], ln_g[l], ln_b[l],
                   block_rows=2000)

    out = _head(x, head_w, head_b, block_rows=2000, row_offset=N)
    return out.reshape(1, N)
